# Initial kernel scaffold; baseline (speedup 1.0000x reference)
import jax
import jax.numpy as jnp
from jax.experimental import pallas as pl


def kernel(x, rnn_states, dones, W_ih0, W_hh0, b_ih0, b_hh0, W_ih1, W_hh1, b_ih1, b_hh1):
    raise NotImplementedError("write your pallas kernel here")


# speedup vs baseline: 3.6968x; 3.6968x over previous
import jax
import jax.numpy as jnp
from jax.experimental import pallas as pl
from jax.experimental.pallas import tpu as pltpu

T, N, D, H, L = 512, 16, 512, 512, 2
TT = 32
GRID = T // TT


def _lstm_chunk_kernel(x_ref, mask_ref, hc0_ref, wih0_ref, whh0_ref, w1_ref,
                       b0_ref, b1_ref, ys_ref, fin_ref, g0_ref):
    i = pl.program_id(0)

    @pl.when(i == 0)
    def _():
        fin_ref[...] = hc0_ref[...]

    xv = x_ref[...].reshape(TT * N, D)
    g0_ref[...] = (
        jnp.dot(xv, wih0_ref[...], preferred_element_type=jnp.float32)
        + b0_ref[...]
    )

    init = (fin_ref[0], fin_ref[2], fin_ref[1], fin_ref[3])

    def step(t, carry):
        h0, c0, h1, c1 = carry
        mt = mask_ref[pl.ds(t, 1)].reshape(N, 128)[:, :1]

        h0m = h0 * mt
        c0m = c0 * mt
        g0 = g0_ref[pl.ds(t * N, N), :] + jnp.dot(
            h0m, whh0_ref[...], preferred_element_type=jnp.float32)
        i0 = jax.nn.sigmoid(g0[:, :H])
        f0 = jax.nn.sigmoid(g0[:, H:2 * H])
        gg0 = jnp.tanh(g0[:, 2 * H:3 * H])
        o0 = jax.nn.sigmoid(g0[:, 3 * H:])
        c0n = f0 * c0m + i0 * gg0
        h0n = o0 * jnp.tanh(c0n)

        h1m = h1 * mt
        c1m = c1 * mt
        inp1 = jnp.concatenate([h0n, h1m], axis=1)
        g1 = jnp.dot(inp1, w1_ref[...],
                     preferred_element_type=jnp.float32) + b1_ref[...]
        i1 = jax.nn.sigmoid(g1[:, :H])
        f1 = jax.nn.sigmoid(g1[:, H:2 * H])
        gg1 = jnp.tanh(g1[:, 2 * H:3 * H])
        o1 = jax.nn.sigmoid(g1[:, 3 * H:])
        c1n = f1 * c1m + i1 * gg1
        h1n = o1 * jnp.tanh(c1n)

        ys_ref[pl.ds(t, 1)] = h1n[None]
        return (h0n, c0n, h1n, c1n)

    h0, c0, h1, c1 = jax.lax.fori_loop(0, TT, step, init)
    fin_ref[0] = h0
    fin_ref[1] = h1
    fin_ref[2] = c0
    fin_ref[3] = c1


def kernel(x, rnn_states, dones, W_ih0, W_hh0, b_ih0, b_hh0,
           W_ih1, W_hh1, b_ih1, b_hh1):
    xs = x.reshape(T, N, D)
    mask_b = jnp.broadcast_to(
        (1.0 - dones.astype(jnp.float32))[:, :, None], (T, N, 128))
    wih0 = W_ih0.T
    whh0 = W_hh0.T
    w1 = jnp.concatenate([W_ih1.T, W_hh1.T], axis=0)
    b0 = (b_ih0 + b_hh0).reshape(1, 4 * H)
    b1 = (b_ih1 + b_hh1).reshape(1, 4 * H)

    full = lambda shape: pl.BlockSpec(shape, lambda i: (0,) * len(shape))

    ys, fin = pl.pallas_call(
        _lstm_chunk_kernel,
        grid=(GRID,),
        in_specs=[
            pl.BlockSpec((TT, N, D), lambda i: (i, 0, 0)),
            pl.BlockSpec((TT, N, 128), lambda i: (i, 0, 0)),
            full((2 * L, N, H)),
            full((D, 4 * H)),
            full((H, 4 * H)),
            full((2 * H, 4 * H)),
            full((1, 4 * H)),
            full((1, 4 * H)),
        ],
        out_specs=[
            pl.BlockSpec((TT, N, H), lambda i: (i, 0, 0)),
            full((2 * L, N, H)),
        ],
        out_shape=[
            jax.ShapeDtypeStruct((T, N, H), jnp.float32),
            jax.ShapeDtypeStruct((2 * L, N, H), jnp.float32),
        ],
        scratch_shapes=[pltpu.VMEM((TT * N, 4 * H), jnp.float32)],
    )(xs, mask_b, rnn_states, wih0, whh0, w1, b0, b1)

    return ys.reshape(T * N, H), fin


# speedup vs baseline: 3.7905x; 1.0253x over previous
import jax
import jax.numpy as jnp
from jax.experimental import pallas as pl
from jax.experimental.pallas import tpu as pltpu

T, N, D, H, L = 512, 16, 512, 512, 2
TT = 32
GRID = T // TT


def _lstm_chunk_kernel(x_ref, mask_ref, hc0_ref, wih0_ref, whh0_ref, w1_ref,
                       b0_ref, b1_ref, ys_ref, fin_ref, g0_ref):
    i = pl.program_id(0)

    @pl.when(i == 0)
    def _():
        fin_ref[...] = hc0_ref[...]

    xv = x_ref[...].reshape(TT * N, D).astype(jnp.bfloat16)
    g0_ref[...] = (
        jnp.dot(xv, wih0_ref[...], preferred_element_type=jnp.float32)
        + b0_ref[...]
    )

    init = (fin_ref[0], fin_ref[2], fin_ref[1], fin_ref[3])

    def step(t, carry):
        h0, c0, h1, c1 = carry
        mt = mask_ref[pl.ds(t, 1)].reshape(N, 128)[:, :1]

        h0m = h0 * mt
        c0m = c0 * mt
        g0 = g0_ref[pl.ds(t * N, N), :] + jnp.dot(
            h0m.astype(jnp.bfloat16), whh0_ref[...],
            preferred_element_type=jnp.float32)
        i0 = jax.nn.sigmoid(g0[:, :H])
        f0 = jax.nn.sigmoid(g0[:, H:2 * H])
        gg0 = jnp.tanh(g0[:, 2 * H:3 * H])
        o0 = jax.nn.sigmoid(g0[:, 3 * H:])
        c0n = f0 * c0m + i0 * gg0
        h0n = o0 * jnp.tanh(c0n)

        h1m = h1 * mt
        c1m = c1 * mt
        inp1 = jnp.concatenate(
            [h0n.astype(jnp.bfloat16), h1m.astype(jnp.bfloat16)],
            axis=1)
        g1 = jnp.dot(inp1, w1_ref[...],
                     preferred_element_type=jnp.float32) + b1_ref[...]
        i1 = jax.nn.sigmoid(g1[:, :H])
        f1 = jax.nn.sigmoid(g1[:, H:2 * H])
        gg1 = jnp.tanh(g1[:, 2 * H:3 * H])
        o1 = jax.nn.sigmoid(g1[:, 3 * H:])
        c1n = f1 * c1m + i1 * gg1
        h1n = o1 * jnp.tanh(c1n)

        ys_ref[pl.ds(t, 1)] = h1n[None]
        return (h0n, c0n, h1n, c1n)

    h0, c0, h1, c1 = jax.lax.fori_loop(0, TT, step, init)
    fin_ref[0] = h0
    fin_ref[1] = h1
    fin_ref[2] = c0
    fin_ref[3] = c1


def kernel(x, rnn_states, dones, W_ih0, W_hh0, b_ih0, b_hh0,
           W_ih1, W_hh1, b_ih1, b_hh1):
    xs = x.reshape(T, N, D)
    mask_b = jnp.broadcast_to(
        (1.0 - dones.astype(jnp.float32))[:, :, None], (T, N, 128))
    wih0 = W_ih0.T.astype(jnp.bfloat16)
    whh0 = W_hh0.T.astype(jnp.bfloat16)
    w1 = jnp.concatenate([W_ih1.T, W_hh1.T],
                         axis=0).astype(jnp.bfloat16)
    b0 = (b_ih0 + b_hh0).reshape(1, 4 * H)
    b1 = (b_ih1 + b_hh1).reshape(1, 4 * H)

    full = lambda shape: pl.BlockSpec(shape, lambda i: (0,) * len(shape))

    ys, fin = pl.pallas_call(
        _lstm_chunk_kernel,
        grid=(GRID,),
        in_specs=[
            pl.BlockSpec((TT, N, D), lambda i: (i, 0, 0)),
            pl.BlockSpec((TT, N, 128), lambda i: (i, 0, 0)),
            full((2 * L, N, H)),
            full((D, 4 * H)),
            full((H, 4 * H)),
            full((2 * H, 4 * H)),
            full((1, 4 * H)),
            full((1, 4 * H)),
        ],
        out_specs=[
            pl.BlockSpec((TT, N, H), lambda i: (i, 0, 0)),
            full((2 * L, N, H)),
        ],
        out_shape=[
            jax.ShapeDtypeStruct((T, N, H), jnp.float32),
            jax.ShapeDtypeStruct((2 * L, N, H), jnp.float32),
        ],
        scratch_shapes=[pltpu.VMEM((TT * N, 4 * H), jnp.float32)],
    )(xs, mask_b, rnn_states, wih0, whh0, w1, b0, b1)

    return ys.reshape(T * N, H), fin


# speedup vs baseline: 4.1426x; 1.0929x over previous
import jax
import jax.numpy as jnp
from jax.experimental import pallas as pl
from jax.experimental.pallas import tpu as pltpu

T, N, D, H, L = 512, 16, 512, 512, 2
TT = 32
GRID = T // TT
TPAD = T + 8


def _lstm_chunk_kernel(x_ref, mask_ref, hc0_ref, wih0_ref, whh0_ref, w1_ref,
                       b0_ref, b1_ref, ys_ref, fin_ref, g0_ref, ma_ref):
    i = pl.program_id(0)
    base = i * TT

    def mrow(idx):
        return mask_ref[pl.ds(idx, 1)].reshape(N, 128)[:, :1]

    @pl.when(i == 0)
    def _():
        fin_ref[...] = hc0_ref[...]
        h0m0 = hc0_ref[0] * mrow(0)
        ma_ref[...] = jnp.dot(h0m0.astype(jnp.bfloat16), whh0_ref[...],
                              preferred_element_type=jnp.float32)

    xv = x_ref[...].reshape(TT * N, D).astype(jnp.bfloat16)
    g0_ref[...] = (
        jnp.dot(xv, wih0_ref[...], preferred_element_type=jnp.float32)
        + b0_ref[...]
    )

    init = (fin_ref[0], fin_ref[2], fin_ref[1], fin_ref[3], ma_ref[...])

    def step(t, carry):
        h0, c0, h1, c1, ma = carry
        mt = mrow(base + t)

        c0m = c0 * mt
        g0 = g0_ref[pl.ds(t * N, N), :] + ma
        i0 = jax.nn.sigmoid(g0[:, :H])
        f0 = jax.nn.sigmoid(g0[:, H:2 * H])
        gg0 = jnp.tanh(g0[:, 2 * H:3 * H])
        o0 = jax.nn.sigmoid(g0[:, 3 * H:])
        c0n = f0 * c0m + i0 * gg0
        h0n = o0 * jnp.tanh(c0n)

        h1m = h1 * mt
        c1m = c1 * mt
        inp1 = jnp.concatenate(
            [h0n.astype(jnp.bfloat16), h1m.astype(jnp.bfloat16)], axis=1)
        g1 = jnp.dot(inp1, w1_ref[...],
                     preferred_element_type=jnp.float32) + b1_ref[...]

        h0m_next = h0n * mrow(base + t + 1)
        ma_next = jnp.dot(h0m_next.astype(jnp.bfloat16), whh0_ref[...],
                          preferred_element_type=jnp.float32)

        i1 = jax.nn.sigmoid(g1[:, :H])
        f1 = jax.nn.sigmoid(g1[:, H:2 * H])
        gg1 = jnp.tanh(g1[:, 2 * H:3 * H])
        o1 = jax.nn.sigmoid(g1[:, 3 * H:])
        c1n = f1 * c1m + i1 * gg1
        h1n = o1 * jnp.tanh(c1n)

        ys_ref[pl.ds(t, 1)] = h1n[None]
        return (h0n, c0n, h1n, c1n, ma_next)

    h0, c0, h1, c1, ma = jax.lax.fori_loop(0, TT, step, init)
    fin_ref[0] = h0
    fin_ref[1] = h1
    fin_ref[2] = c0
    fin_ref[3] = c1
    ma_ref[...] = ma


def kernel(x, rnn_states, dones, W_ih0, W_hh0, b_ih0, b_hh0,
           W_ih1, W_hh1, b_ih1, b_hh1):
    xs = x.reshape(T, N, D)
    mask_b = jnp.zeros((TPAD, N, 128), jnp.float32)
    mask_b = mask_b.at[:T].set(
        jnp.broadcast_to((1.0 - dones.astype(jnp.float32))[:, :, None],
                         (T, N, 128)))
    wih0 = W_ih0.T.astype(jnp.bfloat16)
    whh0 = W_hh0.T.astype(jnp.bfloat16)
    w1 = jnp.concatenate([W_ih1.T, W_hh1.T],
                         axis=0).astype(jnp.bfloat16)
    b0 = (b_ih0 + b_hh0).reshape(1, 4 * H)
    b1 = (b_ih1 + b_hh1).reshape(1, 4 * H)

    full = lambda shape: pl.BlockSpec(shape, lambda i: (0,) * len(shape))

    ys, fin = pl.pallas_call(
        _lstm_chunk_kernel,
        grid=(GRID,),
        in_specs=[
            pl.BlockSpec((TT, N, D), lambda i: (i, 0, 0)),
            full((TPAD, N, 128)),
            full((2 * L, N, H)),
            full((D, 4 * H)),
            full((H, 4 * H)),
            full((2 * H, 4 * H)),
            full((1, 4 * H)),
            full((1, 4 * H)),
        ],
        out_specs=[
            pl.BlockSpec((TT, N, H), lambda i: (i, 0, 0)),
            full((2 * L, N, H)),
        ],
        out_shape=[
            jax.ShapeDtypeStruct((T, N, H), jnp.float32),
            jax.ShapeDtypeStruct((2 * L, N, H), jnp.float32),
        ],
        scratch_shapes=[
            pltpu.VMEM((TT * N, 4 * H), jnp.float32),
            pltpu.VMEM((N, 4 * H), jnp.float32),
        ],
    )(xs, mask_b, rnn_states, wih0, whh0, w1, b0, b1)

    return ys.reshape(T * N, H), fin


# speedup vs baseline: 4.1546x; 1.0029x over previous
import jax
import jax.numpy as jnp
from jax.experimental import pallas as pl
from jax.experimental.pallas import tpu as pltpu

T, N, D, H, L = 512, 16, 512, 512, 2
TT = 64
GRID = T // TT
TPAD = T + 8


def _lstm_chunk_kernel(x_ref, mask_ref, hc0_ref, wih0_ref, whh0_ref, w1_ref,
                       b0_ref, b1_ref, ys_ref, fin_ref, g0_ref, ma_ref):
    i = pl.program_id(0)
    base = i * TT

    def mrow(idx):
        return mask_ref[pl.ds(idx, 1)].reshape(N, 128)[:, :1]

    @pl.when(i == 0)
    def _():
        fin_ref[...] = hc0_ref[...]
        h0m0 = hc0_ref[0] * mrow(0)
        ma_ref[...] = jnp.dot(h0m0.astype(jnp.bfloat16), whh0_ref[...],
                              preferred_element_type=jnp.float32)

    xv = x_ref[...].reshape(TT * N, D).astype(jnp.bfloat16)
    g0_ref[...] = (
        jnp.dot(xv, wih0_ref[...], preferred_element_type=jnp.float32)
        + b0_ref[...]
    )

    init = (fin_ref[0], fin_ref[2], fin_ref[1], fin_ref[3], ma_ref[...])

    def step(t, carry):
        h0, c0, h1, c1, ma = carry
        mt = mrow(base + t)

        c0m = c0 * mt
        g0 = g0_ref[pl.ds(t * N, N), :] + ma
        i0 = jax.nn.sigmoid(g0[:, :H])
        f0 = jax.nn.sigmoid(g0[:, H:2 * H])
        gg0 = jnp.tanh(g0[:, 2 * H:3 * H])
        o0 = jax.nn.sigmoid(g0[:, 3 * H:])
        c0n = f0 * c0m + i0 * gg0
        h0n = o0 * jnp.tanh(c0n)

        h1m = h1 * mt
        c1m = c1 * mt
        inp1 = jnp.concatenate(
            [h0n.astype(jnp.bfloat16), h1m.astype(jnp.bfloat16)], axis=1)
        g1 = jnp.dot(inp1, w1_ref[...],
                     preferred_element_type=jnp.float32) + b1_ref[...]

        h0m_next = h0n * mrow(base + t + 1)
        ma_next = jnp.dot(h0m_next.astype(jnp.bfloat16), whh0_ref[...],
                          preferred_element_type=jnp.float32)

        i1 = jax.nn.sigmoid(g1[:, :H])
        f1 = jax.nn.sigmoid(g1[:, H:2 * H])
        gg1 = jnp.tanh(g1[:, 2 * H:3 * H])
        o1 = jax.nn.sigmoid(g1[:, 3 * H:])
        c1n = f1 * c1m + i1 * gg1
        h1n = o1 * jnp.tanh(c1n)

        ys_ref[pl.ds(t, 1)] = h1n[None]
        return (h0n, c0n, h1n, c1n, ma_next)

    h0, c0, h1, c1, ma = jax.lax.fori_loop(0, TT, step, init)
    fin_ref[0] = h0
    fin_ref[1] = h1
    fin_ref[2] = c0
    fin_ref[3] = c1
    ma_ref[...] = ma


def kernel(x, rnn_states, dones, W_ih0, W_hh0, b_ih0, b_hh0,
           W_ih1, W_hh1, b_ih1, b_hh1):
    xs = x.reshape(T, N, D)
    mask_b = jnp.zeros((TPAD, N, 128), jnp.float32)
    mask_b = mask_b.at[:T].set(
        jnp.broadcast_to((1.0 - dones.astype(jnp.float32))[:, :, None],
                         (T, N, 128)))
    wih0 = W_ih0.T.astype(jnp.bfloat16)
    whh0 = W_hh0.T.astype(jnp.bfloat16)
    w1 = jnp.concatenate([W_ih1.T, W_hh1.T],
                         axis=0).astype(jnp.bfloat16)
    b0 = (b_ih0 + b_hh0).reshape(1, 4 * H)
    b1 = (b_ih1 + b_hh1).reshape(1, 4 * H)

    full = lambda shape: pl.BlockSpec(shape, lambda i: (0,) * len(shape))

    ys, fin = pl.pallas_call(
        _lstm_chunk_kernel,
        grid=(GRID,),
        in_specs=[
            pl.BlockSpec((TT, N, D), lambda i: (i, 0, 0)),
            full((TPAD, N, 128)),
            full((2 * L, N, H)),
            full((D, 4 * H)),
            full((H, 4 * H)),
            full((2 * H, 4 * H)),
            full((1, 4 * H)),
            full((1, 4 * H)),
        ],
        out_specs=[
            pl.BlockSpec((TT, N, H), lambda i: (i, 0, 0)),
            full((2 * L, N, H)),
        ],
        out_shape=[
            jax.ShapeDtypeStruct((T, N, H), jnp.float32),
            jax.ShapeDtypeStruct((2 * L, N, H), jnp.float32),
        ],
        scratch_shapes=[
            pltpu.VMEM((TT * N, 4 * H), jnp.float32),
            pltpu.VMEM((N, 4 * H), jnp.float32),
        ],
    )(xs, mask_b, rnn_states, wih0, whh0, w1, b0, b1)

    return ys.reshape(T * N, H), fin
